# baseline (speedup 1.0000x reference)
import functools
import jax
import jax.numpy as jnp
from jax import lax
from jax.experimental import pallas as pl
from jax.experimental.pallas import tpu as pltpu, tpu_sc as plsc

_info = plsc.get_sparse_core_info()
_NC, _NS = _info.num_cores, _info.num_subcores


def _make_lookup(B: int, V: int, D: int, r0: int):
    r1 = B // _NS - r0
    assert r0 % 16 == 0 and r1 % 16 == 0 and _NC == 2
    mesh = plsc.VectorSubcoreMesh(core_axis_name="c", subcore_axis_name="s")

    @functools.partial(
        pl.kernel,
        mesh=mesh,
        compiler_params=pltpu.CompilerParams(needs_layout_passes=False),
        out_type=jax.ShapeDtypeStruct((B, D), jnp.float32),
        scratch_types=[
            pltpu.VMEM((V, D), jnp.float32),
            pltpu.VMEM((max(r0, r1),), jnp.int32),
            pltpu.SemaphoreType.DMA,
        ],
    )
    def lookup(table_hbm, idx_hbm, out_hbm, table_v, idx_v, sem):
        core = lax.axis_index("c")
        sub = lax.axis_index("s")
        pltpu.sync_copy(table_hbm, table_v)

        def run(base, rows):
            pltpu.sync_copy(idx_hbm.at[pl.ds(base, rows)], idx_v.at[pl.ds(0, rows)])

            @pl.loop(0, rows // 16)
            def _(g):
                vec = idx_v[pl.ds(g * 16, 16)]
                for j in range(16):
                    row = vec[j]
                    pltpu.async_copy(
                        table_v.at[row], out_hbm.at[base + g * 16 + j], sem
                    )

            @pl.loop(0, rows)
            def _(r):
                pltpu.make_async_copy(out_hbm.at[0], table_v.at[0], sem).wait()

        @pl.when(core == 0)
        def _():
            run(sub * r0, r0)

        @pl.when(core == 1)
        def _():
            run(_NS * r0 + sub * r1, r1)

    return lookup


def kernel(placeholder, table):
    B = placeholder.size
    V, D = table.shape
    idx = placeholder.reshape(B).astype(jnp.int32)
    out = _make_lookup(B, V, D, r0=928)(table, idx)
    return out.reshape(*placeholder.shape, D)


# speedup vs baseline: 1.0159x; 1.0159x over previous
import functools
import jax
import jax.numpy as jnp
from jax import lax
from jax.experimental import pallas as pl
from jax.experimental.pallas import tpu as pltpu, tpu_sc as plsc

_info = plsc.get_sparse_core_info()
_NC, _NS = _info.num_cores, _info.num_subcores


def _make_lookup(B: int, V: int, D: int, r0: int):
    r1 = B // _NS - r0
    assert r0 % 16 == 0 and r1 % 16 == 0 and _NC == 2
    mesh = plsc.VectorSubcoreMesh(core_axis_name="c", subcore_axis_name="s")

    @functools.partial(
        pl.kernel,
        mesh=mesh,
        compiler_params=pltpu.CompilerParams(needs_layout_passes=False),
        out_type=jax.ShapeDtypeStruct((B, D), jnp.float32),
        scratch_types=[
            pltpu.VMEM((V, D), jnp.float32),
            pltpu.VMEM((max(r0, r1),), jnp.int32),
            pltpu.SemaphoreType.DMA,
        ],
    )
    def lookup(table_hbm, idx_hbm, out_hbm, table_v, idx_v, sem):
        core = lax.axis_index("c")
        sub = lax.axis_index("s")
        pltpu.sync_copy(table_hbm, table_v)

        def run(base, rows):
            pltpu.sync_copy(idx_hbm.at[pl.ds(base, rows)], idx_v.at[pl.ds(0, rows)])

            @pl.loop(0, rows // 16)
            def _(g):
                vec = idx_v[pl.ds(g * 16, 16)]
                for j in range(16):
                    row = vec[j]
                    pltpu.async_copy(
                        table_v.at[row], out_hbm.at[base + g * 16 + j], sem
                    )

            @pl.loop(0, rows)
            def _(r):
                pltpu.make_async_copy(out_hbm.at[0], table_v.at[0], sem).wait()

        @pl.when(core == 0)
        def _():
            run(sub * r0, r0)

        @pl.when(core == 1)
        def _():
            run(_NS * r0 + sub * r1, r1)

    return lookup


def kernel(placeholder, table):
    B = placeholder.size
    V, D = table.shape
    idx = placeholder.reshape(B).astype(jnp.int32)
    out = _make_lookup(B, V, D, r0=992)(table, idx)
    return out.reshape(*placeholder.shape, D)
